# Initial kernel scaffold; baseline (speedup 1.0000x reference)
import jax
import jax.numpy as jnp
from jax.experimental import pallas as pl


def kernel(atom_features, bond_features, atom_neighbor_list, bond_neighbor_list, atom_mask, params):
    raise NotImplementedError("write your pallas kernel here")


# speedup vs baseline: 6.0953x; 6.0953x over previous
import jax
import jax.numpy as jnp
from jax.experimental import pallas as pl

B, A, K, MB = 256, 64, 6, 64
AF, BF, FP = 72, 10, 128
NLAYERS, NMOL = 3, 2
EPS = 1e-06
NEG = -900000000.0
MBLK = 8
N = MBLK * A
_SQ = float((1.0 + EPS) ** 0.5)


def _fold(p):
    s = p['g'] / _SQ
    w_eff = p['W'].T * s[None, :]
    b_eff = (p['b'] * s + p['be']).reshape(1, -1)
    return w_eff, b_eff


def _leaky(x):
    return jnp.where(x >= 0, x, 0.01 * x)


def _elu(x):
    return jnp.where(x > 0, x, jnp.exp(jnp.minimum(x, 0.0)) - 1.0)


def _body(af_ref, bf_ref, ia_ref, ib_ref, ik_ref, mk_ref, *refs):
    o_ref = refs[-1]
    w = [r[...] for r in refs[:-1]]
    (Wa1, ba1, Wa2, ba2, Wb1, bb1, Wb2, bb2, Wn1, bn1, Wn2, bn2) = w[:12]
    layer_w = [w[12 + 9 * i: 12 + 9 * (i + 1)] for i in range(NLAYERS)]
    mol_w = w[12 + 9 * NLAYERS: 12 + 9 * NLAYERS + 9]
    (W1d, b1d, W2d, b2d, W3d, b3d) = w[12 + 9 * NLAYERS + 9:]

    f32 = jnp.float32
    dot = lambda a, b_: jnp.dot(a, b_, preferred_element_type=f32,
                                precision=jax.lax.Precision.HIGHEST)

    x = af_ref[...].reshape(N, AF)
    x = jnp.maximum(dot(x, Wa1) + ba1, 0.0)
    atom_fp = jnp.maximum(dot(x, Wa2) + ba2, 0.0)
    y = bf_ref[...].reshape(N, BF)
    y = jnp.maximum(dot(y, Wb1) + bb1, 0.0)
    bond_fp = jnp.maximum(dot(y, Wb2) + bb2, 0.0)

    iota = jax.lax.broadcasted_iota(jnp.int32, (K * A, A), 1)

    def gather_km(idx_all, table):
        outs = [[] for _ in range(K)]
        for m in range(MBLK):
            oh = (idx_all[m] == iota).astype(f32)
            g = dot(oh, table[m * A:(m + 1) * A])
            for k in range(K):
                outs[k].append(g[k * A:(k + 1) * A])
        return [jnp.concatenate(c, axis=0) for c in outs]

    ia = ia_ref[...]
    ib = ib_ref[...]
    anei = gather_km(ia, atom_fp)
    bnei = gather_km(ib, bond_fp)

    nei = []
    for k in range(K):
        a_k, b_k = anei[k], bnei[k]
        mix = a_k + b_k - a_k * b_k
        nb = jnp.concatenate([a_k, b_k, mix], axis=1)
        h1 = jnp.maximum(dot(nb, Wn1) + bn1, 0.0)
        nei.append(jnp.maximum(dot(h1, Wn2) + bn2, 0.0))

    idxs = ik_ref[...].reshape(N, K)
    att_mask = (idxs != A - 1).astype(f32)
    smask = jnp.where(idxs == A - 1, NEG, 0.0).astype(f32)

    for (wal, Wbd, bal, Watt, batt, Wih, bih, Whh, bhh) in layer_w:
        nei_stack = jnp.concatenate(nei, axis=1)
        s = dot(atom_fp, wal) + dot(nei_stack, Wbd)
        s = _leaky(s + bal) + smask
        s = s - jnp.max(s, axis=1, keepdims=True)
        e = jnp.exp(s)
        wgt = e / jnp.sum(e, axis=1, keepdims=True) * att_mask
        wsum = jnp.sum(wgt, axis=1, keepdims=True)
        weighted = wgt[:, 0:1] * nei[0]
        for k in range(1, K):
            weighted = weighted + wgt[:, k:k + 1] * nei[k]
        ctx = _elu(dot(weighted, Watt) + wsum * batt)
        gi = dot(ctx, Wih) + bih
        gh = dot(atom_fp, Whh) + bhh
        r = jax.nn.sigmoid(gi[:, :FP] + gh[:, :FP])
        z = jax.nn.sigmoid(gi[:, FP:2 * FP] + gh[:, FP:2 * FP])
        n = jnp.tanh(gi[:, 2 * FP:] + r * gh[:, 2 * FP:])
        new_atom = (1.0 - z) * n + z * atom_fp
        act = jnp.maximum(new_atom, 0.0)
        nei = gather_km(ia, act)
        atom_fp = new_atom

    mask3 = mk_ref[...]
    molmask = jnp.where(mask3 == 0.0, NEG, 0.0)
    afp3 = atom_fp.reshape(MBLK, A, FP)
    sup = jnp.sum(afp3 * mask3, axis=1)
    (wsm, wam, bam, Wam2, bam2, Wihm, bihm, Whhm, bhhm) = mol_w
    for _ in range(NMOL):
        s_at = dot(atom_fp, wam).reshape(MBLK, A, 1)
        s_sup = dot(sup, wsm).reshape(MBLK, 1, 1)
        s = _leaky(s_at + s_sup + bam) + molmask
        s = s - jnp.max(s, axis=1, keepdims=True)
        e = jnp.exp(s)
        wgt = e / jnp.sum(e, axis=1, keepdims=True) * mask3
        att = dot(atom_fp, Wam2) + bam2
        ctx = _elu(jnp.sum(att.reshape(MBLK, A, FP) * wgt, axis=1))
        gi = dot(ctx, Wihm) + bihm
        gh = dot(sup, Whhm) + bhhm
        r = jax.nn.sigmoid(gi[:, :FP] + gh[:, :FP])
        z = jax.nn.sigmoid(gi[:, FP:2 * FP] + gh[:, FP:2 * FP])
        n = jnp.tanh(gi[:, 2 * FP:] + r * gh[:, 2 * FP:])
        sup = (1.0 - z) * n + z * sup

    mol_fp = jnp.maximum(sup, 0.0)
    h = jnp.maximum(dot(mol_fp, W1d) + b1d, 0.0)
    h = jnp.maximum(dot(h, W2d) + b2d, 0.0)
    o_ref[...] = dot(h, W3d) + b3d


def kernel(atom_features, bond_features, atom_neighbor_list,
           bond_neighbor_list, atom_mask, params):
    f32 = jnp.float32
    Wa1, ba1 = _fold(params['atom_fc'][0])
    Wa2, ba2 = _fold(params['atom_fc'][1])
    Wb1, bb1 = _fold(params['bond_fc'][0])
    Wb2, bb2 = _fold(params['bond_fc'][1])
    Wn1, bn1 = _fold(params['nei_fc'][0])
    Wn2, bn2 = _fold(params['nei_fc'][1])
    wlist = [Wa1, ba1, Wa2, ba2, Wb1, bb1, Wb2, bb2, Wn1, bn1, Wn2, bn2]
    eye_k = jnp.eye(K, dtype=f32)
    for lp in params['layers']:
        wal_full, bal = _fold(lp['align'])
        wal = wal_full[:FP]
        wn = wal_full[FP:]
        Wbd = jnp.kron(eye_k, wn)
        Watt, batt = _fold(lp['attend'])
        g = lp['gru']
        wlist += [wal, Wbd, bal, Watt, batt,
                  g['Wih'].T, g['bih'].reshape(1, -1),
                  g['Whh'].T, g['bhh'].reshape(1, -1)]
    mp = params['mol']
    wm_full, bam = _fold(mp['align'])
    Wam2, bam2 = _fold(mp['attend'])
    g = mp['gru']
    wlist += [wm_full[:FP], wm_full[FP:], bam, Wam2, bam2,
              g['Wih'].T, g['bih'].reshape(1, -1),
              g['Whh'].T, g['bhh'].reshape(1, -1)]
    d = params['dnn']
    wlist += [d['W1'].T, d['b1'].reshape(1, -1),
              d['W2'].T, d['b2'].reshape(1, -1),
              d['W3'].T, d['b3'].reshape(1, -1)]
    wlist = [w.astype(f32) for w in wlist]

    ia = jnp.transpose(atom_neighbor_list, (0, 2, 1)).reshape(B, K * A, 1)
    ib = jnp.transpose(bond_neighbor_list, (0, 2, 1)).reshape(B, K * A, 1)
    mask3 = atom_mask[..., None]

    in_specs = [
        pl.BlockSpec((MBLK, A, AF), lambda i: (i, 0, 0)),
        pl.BlockSpec((MBLK, MB, BF), lambda i: (i, 0, 0)),
        pl.BlockSpec((MBLK, K * A, 1), lambda i: (i, 0, 0)),
        pl.BlockSpec((MBLK, K * A, 1), lambda i: (i, 0, 0)),
        pl.BlockSpec((MBLK, A, K), lambda i: (i, 0, 0)),
        pl.BlockSpec((MBLK, A, 1), lambda i: (i, 0, 0)),
    ] + [pl.BlockSpec(wa.shape, lambda i, nd=wa.ndim: (0,) * nd)
         for wa in wlist]

    return pl.pallas_call(
        _body,
        grid=(B // MBLK,),
        in_specs=in_specs,
        out_specs=pl.BlockSpec((MBLK, 1), lambda i: (i, 0)),
        out_shape=jax.ShapeDtypeStruct((B, 1), f32),
    )(atom_features, bond_features, ia, ib, atom_neighbor_list, mask3,
      *wlist)


# speedup vs baseline: 14.2854x; 2.3437x over previous
import jax
import jax.numpy as jnp
from jax.experimental import pallas as pl

B, A, K, MB = 256, 64, 6, 64
AF, BF, FP = 72, 10, 128
NLAYERS, NMOL = 3, 2
EPS = 1e-06
NEG = -900000000.0
MBLK = 8
N = MBLK * A
_SQ = float((1.0 + EPS) ** 0.5)


def _lbn_w(p):
    return [p['W'].T, p['b'].reshape(1, -1),
            (p['g'] / _SQ).reshape(1, -1), p['be'].reshape(1, -1)]


def _leaky(x):
    return jnp.where(x >= 0, x, 0.01 * x)


def _elu(x):
    return jnp.where(x > 0, x, jnp.exp(jnp.minimum(x, 0.0)) - 1.0)


def _body(af_ref, bf_ref, ia_ref, ib_ref, ik_ref, mk_ref, *refs):
    o_ref = refs[-1]
    w = [r[...] for r in refs[:-1]]
    it = iter(w)

    def nxt(n_):
        return [next(it) for _ in range(n_)]

    f32 = jnp.float32
    bf16 = jnp.bfloat16
    dot = lambda a, b_: jnp.dot(a, b_, preferred_element_type=f32)

    def lbn(x, ws):
        Wt, b_, s_, be_ = ws
        return (dot(x, Wt) + b_) * s_ + be_

    x = af_ref[...].reshape(N, AF)
    x = jnp.maximum(lbn(x, nxt(4)), 0.0)
    atom_fp = jnp.maximum(lbn(x, nxt(4)), 0.0)
    y = bf_ref[...].reshape(N, BF)
    y = jnp.maximum(lbn(y, nxt(4)), 0.0)
    bond_fp = jnp.maximum(lbn(y, nxt(4)), 0.0)

    iota = jax.lax.broadcasted_iota(jnp.int32, (K * A, A), 1)

    def build_oh(idx_all):
        ohs = []
        for m in range(MBLK):
            oh = (idx_all[m] == iota).astype(bf16)
            ohs.append(jnp.concatenate([oh, oh], axis=1))
        return ohs

    def gather_km(ohs, table):
        hi = table.astype(bf16)
        lo = (table - hi.astype(f32)).astype(bf16)
        outs = [[] for _ in range(K)]
        for m in range(MBLK):
            t2 = jnp.concatenate(
                [hi[m * A:(m + 1) * A], lo[m * A:(m + 1) * A]], axis=0)
            g = jnp.dot(ohs[m], t2, preferred_element_type=f32)
            for k in range(K):
                outs[k].append(g[k * A:(k + 1) * A])
        return [jnp.concatenate(c, axis=0) for c in outs]

    oh_a = build_oh(ia_ref[...])
    oh_b = build_oh(ib_ref[...])
    anei = gather_km(oh_a, atom_fp)
    bnei = gather_km(oh_b, bond_fp)

    nei_w1, nei_w2 = nxt(4), nxt(4)
    nei = []
    for k in range(K):
        a_k, b_k = anei[k], bnei[k]
        mix = a_k + b_k - a_k * b_k
        nb = jnp.concatenate([a_k, b_k, mix], axis=1)
        h1 = jnp.maximum(lbn(nb, nei_w1), 0.0)
        nei.append(jnp.maximum(lbn(h1, nei_w2), 0.0))

    idxs = ik_ref[...].reshape(N, K)
    att_mask = (idxs != A - 1).astype(f32)
    smask = jnp.where(idxs == A - 1, NEG, 0.0).astype(f32)

    for _ in range(NLAYERS):
        (wal, Wbd, bal, sal, beal) = nxt(5)
        att_w = nxt(4)
        (Wih, bih, Whh, bhh) = nxt(4)
        nei_stack = jnp.concatenate(nei, axis=1)
        s = dot(atom_fp, wal) + dot(nei_stack, Wbd)
        s = (s + bal) * sal + beal
        s = _leaky(s) + smask
        s = s - jnp.max(s, axis=1, keepdims=True)
        e = jnp.exp(s)
        wgt = e / jnp.sum(e, axis=1, keepdims=True) * att_mask
        wsum = jnp.sum(wgt, axis=1, keepdims=True)
        weighted = wgt[:, 0:1] * nei[0]
        for k in range(1, K):
            weighted = weighted + wgt[:, k:k + 1] * nei[k]
        Wt_a, b_a, s_a, be_a = att_w
        ctx = (dot(weighted, Wt_a) + wsum * b_a) * s_a + wsum * be_a
        ctx = _elu(ctx)
        gi = dot(ctx, Wih) + bih
        gh = dot(atom_fp, Whh) + bhh
        r = jax.nn.sigmoid(gi[:, :FP] + gh[:, :FP])
        z = jax.nn.sigmoid(gi[:, FP:2 * FP] + gh[:, FP:2 * FP])
        n = jnp.tanh(gi[:, 2 * FP:] + r * gh[:, 2 * FP:])
        new_atom = (1.0 - z) * n + z * atom_fp
        act = jnp.maximum(new_atom, 0.0)
        nei = gather_km(oh_a, act)
        atom_fp = new_atom

    mask3 = mk_ref[...]
    molmask = jnp.where(mask3 == 0.0, NEG, 0.0)
    afp3 = atom_fp.reshape(MBLK, A, FP)
    sup = jnp.sum(afp3 * mask3, axis=1)
    (wsm, wam, bam, sam, beam) = nxt(5)
    matt_w = nxt(4)
    (Wihm, bihm, Whhm, bhhm) = nxt(4)
    for _ in range(NMOL):
        s_at = dot(atom_fp, wam).reshape(MBLK, A, 1)
        s_sup = dot(sup, wsm).reshape(MBLK, 1, 1)
        s = (s_at + s_sup + bam) * sam + beam
        s = _leaky(s) + molmask
        s = s - jnp.max(s, axis=1, keepdims=True)
        e = jnp.exp(s)
        wgt = e / jnp.sum(e, axis=1, keepdims=True) * mask3
        att = lbn(atom_fp, matt_w)
        ctx = _elu(jnp.sum(att.reshape(MBLK, A, FP) * wgt, axis=1))
        gi = dot(ctx, Wihm) + bihm
        gh = dot(sup, Whhm) + bhhm
        r = jax.nn.sigmoid(gi[:, :FP] + gh[:, :FP])
        z = jax.nn.sigmoid(gi[:, FP:2 * FP] + gh[:, FP:2 * FP])
        n = jnp.tanh(gi[:, 2 * FP:] + r * gh[:, 2 * FP:])
        sup = (1.0 - z) * n + z * sup

    (W1d, b1d, W2d, b2d, W3d, b3d) = nxt(6)
    mol_fp = jnp.maximum(sup, 0.0)
    h = jnp.maximum(dot(mol_fp, W1d) + b1d, 0.0)
    h = jnp.maximum(dot(h, W2d) + b2d, 0.0)
    o_ref[...] = dot(h, W3d) + b3d


def kernel(atom_features, bond_features, atom_neighbor_list,
           bond_neighbor_list, atom_mask, params):
    f32 = jnp.float32
    wlist = []
    for p in params['atom_fc'] + params['bond_fc']:
        wlist += _lbn_w(p)
    for p in params['nei_fc']:
        wlist += _lbn_w(p)
    eye_k = jnp.eye(K, dtype=f32)
    for lp in params['layers']:
        al = lp['align']
        wt = al['W'].T
        wlist += [wt[:FP], jnp.kron(eye_k, wt[FP:]),
                  al['b'].reshape(1, 1), (al['g'] / _SQ).reshape(1, 1),
                  al['be'].reshape(1, 1)]
        wlist += _lbn_w(lp['attend'])
        g = lp['gru']
        wlist += [g['Wih'].T, g['bih'].reshape(1, -1),
                  g['Whh'].T, g['bhh'].reshape(1, -1)]
    mp = params['mol']
    al = mp['align']
    wt = al['W'].T
    wlist += [wt[:FP], wt[FP:], al['b'].reshape(1, 1),
              (al['g'] / _SQ).reshape(1, 1), al['be'].reshape(1, 1)]
    wlist += _lbn_w(mp['attend'])
    g = mp['gru']
    wlist += [g['Wih'].T, g['bih'].reshape(1, -1),
              g['Whh'].T, g['bhh'].reshape(1, -1)]
    d = params['dnn']
    wlist += [d['W1'].T, d['b1'].reshape(1, -1),
              d['W2'].T, d['b2'].reshape(1, -1),
              d['W3'].T, d['b3'].reshape(1, -1)]
    wlist = [w.astype(f32) for w in wlist]

    ia = jnp.transpose(atom_neighbor_list, (0, 2, 1)).reshape(B, K * A, 1)
    ib = jnp.transpose(bond_neighbor_list, (0, 2, 1)).reshape(B, K * A, 1)
    mask3 = atom_mask[..., None]

    in_specs = [
        pl.BlockSpec((MBLK, A, AF), lambda i: (i, 0, 0)),
        pl.BlockSpec((MBLK, MB, BF), lambda i: (i, 0, 0)),
        pl.BlockSpec((MBLK, K * A, 1), lambda i: (i, 0, 0)),
        pl.BlockSpec((MBLK, K * A, 1), lambda i: (i, 0, 0)),
        pl.BlockSpec((MBLK, A, K), lambda i: (i, 0, 0)),
        pl.BlockSpec((MBLK, A, 1), lambda i: (i, 0, 0)),
    ] + [pl.BlockSpec(wa.shape, lambda i, nd=wa.ndim: (0,) * nd)
         for wa in wlist]

    return pl.pallas_call(
        _body,
        grid=(B // MBLK,),
        in_specs=in_specs,
        out_specs=pl.BlockSpec((MBLK, 1), lambda i: (i, 0)),
        out_shape=jax.ShapeDtypeStruct((B, 1), f32),
    )(atom_features, bond_features, ia, ib, atom_neighbor_list, mask3,
      *wlist)


# speedup vs baseline: 15.9136x; 1.1140x over previous
import jax
import jax.numpy as jnp
from jax.experimental import pallas as pl

B, A, K, MB = 256, 64, 6, 64
AF, BF, FP = 72, 10, 128
NLAYERS, NMOL = 3, 2
EPS = 1e-06
NEG = -900000000.0
MBLK = 8
N = MBLK * A
_SQ = float((1.0 + EPS) ** 0.5)


def _lbn_w(p):
    return [p['W'].T, p['b'].reshape(1, -1),
            (p['g'] / _SQ).reshape(1, -1), p['be'].reshape(1, -1)]


def _leaky(x):
    return jnp.where(x >= 0, x, 0.01 * x)


def _elu(x):
    return jnp.where(x > 0, x, jnp.exp(jnp.minimum(x, 0.0)) - 1.0)


def _body(af_ref, bf_ref, ia_ref, ib_ref, ik_ref, mk_ref, *refs):
    o_ref = refs[-1]
    w = [r[...] for r in refs[:-1]]
    it = iter(w)

    def nxt(n_):
        return [next(it) for _ in range(n_)]

    f32 = jnp.float32
    bf16 = jnp.bfloat16
    dot = lambda a, b_: jnp.dot(a, b_, preferred_element_type=f32)

    def lbn(x, ws):
        Wt, b_, s_, be_ = ws
        return (dot(x, Wt) + b_) * s_ + be_

    x = af_ref[...].reshape(N, AF)
    x = jnp.maximum(lbn(x, nxt(4)), 0.0)
    atom_fp = jnp.maximum(lbn(x, nxt(4)), 0.0)
    y = bf_ref[...].reshape(N, BF)
    y = jnp.maximum(lbn(y, nxt(4)), 0.0)
    bond_fp = jnp.maximum(lbn(y, nxt(4)), 0.0)

    iota2 = jax.lax.broadcasted_iota(jnp.int32, (K * A, 2 * A), 1) & (A - 1)

    def build_oh(idx_all):
        return [(idx_all[m] == iota2).astype(bf16) for m in range(MBLK)]

    def gather_km(ohs, table):
        hi = table.astype(bf16)
        lo = (table - hi.astype(f32)).astype(bf16)
        outs = [[] for _ in range(K)]
        for m in range(MBLK):
            t2 = jnp.concatenate(
                [hi[m * A:(m + 1) * A], lo[m * A:(m + 1) * A]], axis=0)
            g = jnp.dot(ohs[m], t2, preferred_element_type=f32)
            for k in range(K):
                outs[k].append(g[k * A:(k + 1) * A])
        return [jnp.concatenate(c, axis=0) for c in outs]

    oh_a = build_oh(ia_ref[...])
    oh_b = build_oh(ib_ref[...])
    anei = gather_km(oh_a, atom_fp)
    bnei = gather_km(oh_b, bond_fp)

    nei_w1, nei_w2 = nxt(4), nxt(4)
    nei = []
    for k in range(K):
        a_k, b_k = anei[k], bnei[k]
        mix = a_k + b_k - a_k * b_k
        nb = jnp.concatenate([a_k, b_k, mix], axis=1)
        h1 = jnp.maximum(lbn(nb, nei_w1), 0.0)
        nei.append(jnp.maximum(lbn(h1, nei_w2), 0.0))

    row_k = jax.lax.broadcasted_iota(jnp.int32, (2 * K, K * FP), 0) % K
    col_k = jax.lax.broadcasted_iota(jnp.int32, (2 * K, K * FP), 1) // FP
    rep_mat = (row_k == col_k).astype(bf16)

    idxs = ik_ref[...].reshape(N, K)
    att_mask = (idxs != A - 1).astype(f32)
    smask = jnp.where(idxs == A - 1, NEG, 0.0).astype(f32)

    for _ in range(NLAYERS):
        (wal, Wbd, bal, sal, beal) = nxt(5)
        att_w = nxt(4)
        (Wih, bih, Whh, bhh) = nxt(4)
        nei_stack = jnp.concatenate(nei, axis=1)
        s = dot(atom_fp, wal) + dot(nei_stack, Wbd)
        s = (s + bal) * sal + beal
        s = _leaky(s) + smask
        s = s - jnp.max(s, axis=1, keepdims=True)
        e = jnp.exp(s)
        wgt = e / jnp.sum(e, axis=1, keepdims=True) * att_mask
        wsum = jnp.sum(wgt, axis=1, keepdims=True)
        whi = wgt.astype(bf16)
        wlo = (wgt - whi.astype(f32)).astype(bf16)
        wrep = jnp.dot(jnp.concatenate([whi, wlo], axis=1), rep_mat,
                       preferred_element_type=f32)
        weighted = wrep[:, :FP] * nei[0]
        for k in range(1, K):
            weighted = weighted + wrep[:, k * FP:(k + 1) * FP] * nei[k]
        Wt_a, b_a, s_a, be_a = att_w
        ctx = (dot(weighted, Wt_a) + wsum * b_a) * s_a + wsum * be_a
        ctx = _elu(ctx)
        gi = dot(ctx, Wih) + bih
        gh = dot(atom_fp, Whh) + bhh
        r = jax.nn.sigmoid(gi[:, :FP] + gh[:, :FP])
        z = jax.nn.sigmoid(gi[:, FP:2 * FP] + gh[:, FP:2 * FP])
        n = jnp.tanh(gi[:, 2 * FP:] + r * gh[:, 2 * FP:])
        new_atom = (1.0 - z) * n + z * atom_fp
        act = jnp.maximum(new_atom, 0.0)
        nei = gather_km(oh_a, act)
        atom_fp = new_atom

    mask3 = mk_ref[...]
    molmask = jnp.where(mask3 == 0.0, NEG, 0.0)
    afp3 = atom_fp.reshape(MBLK, A, FP)
    sup = jnp.sum(afp3 * mask3, axis=1)
    (wsm, wam, bam, sam, beam) = nxt(5)
    matt_w = nxt(4)
    (Wihm, bihm, Whhm, bhhm) = nxt(4)
    for _ in range(NMOL):
        s_at = dot(atom_fp, wam).reshape(MBLK, A, 1)
        s_sup = dot(sup, wsm).reshape(MBLK, 1, 1)
        s = (s_at + s_sup + bam) * sam + beam
        s = _leaky(s) + molmask
        s = s - jnp.max(s, axis=1, keepdims=True)
        e = jnp.exp(s)
        wgt = e / jnp.sum(e, axis=1, keepdims=True) * mask3
        att = lbn(atom_fp, matt_w)
        ctx = _elu(jnp.sum(att.reshape(MBLK, A, FP) * wgt, axis=1))
        gi = dot(ctx, Wihm) + bihm
        gh = dot(sup, Whhm) + bhhm
        r = jax.nn.sigmoid(gi[:, :FP] + gh[:, :FP])
        z = jax.nn.sigmoid(gi[:, FP:2 * FP] + gh[:, FP:2 * FP])
        n = jnp.tanh(gi[:, 2 * FP:] + r * gh[:, 2 * FP:])
        sup = (1.0 - z) * n + z * sup

    (W1d, b1d, W2d, b2d, W3d, b3d) = nxt(6)
    mol_fp = jnp.maximum(sup, 0.0)
    h = jnp.maximum(dot(mol_fp, W1d) + b1d, 0.0)
    h = jnp.maximum(dot(h, W2d) + b2d, 0.0)
    o_ref[...] = dot(h, W3d) + b3d


def kernel(atom_features, bond_features, atom_neighbor_list,
           bond_neighbor_list, atom_mask, params):
    f32 = jnp.float32
    wlist = []
    for p in params['atom_fc'] + params['bond_fc']:
        wlist += _lbn_w(p)
    for p in params['nei_fc']:
        wlist += _lbn_w(p)
    eye_k = jnp.eye(K, dtype=f32)
    for lp in params['layers']:
        al = lp['align']
        wt = al['W'].T
        wlist += [wt[:FP], jnp.kron(eye_k, wt[FP:]),
                  al['b'].reshape(1, 1), (al['g'] / _SQ).reshape(1, 1),
                  al['be'].reshape(1, 1)]
        wlist += _lbn_w(lp['attend'])
        g = lp['gru']
        wlist += [g['Wih'].T, g['bih'].reshape(1, -1),
                  g['Whh'].T, g['bhh'].reshape(1, -1)]
    mp = params['mol']
    al = mp['align']
    wt = al['W'].T
    wlist += [wt[:FP], wt[FP:], al['b'].reshape(1, 1),
              (al['g'] / _SQ).reshape(1, 1), al['be'].reshape(1, 1)]
    wlist += _lbn_w(mp['attend'])
    g = mp['gru']
    wlist += [g['Wih'].T, g['bih'].reshape(1, -1),
              g['Whh'].T, g['bhh'].reshape(1, -1)]
    d = params['dnn']
    wlist += [d['W1'].T, d['b1'].reshape(1, -1),
              d['W2'].T, d['b2'].reshape(1, -1),
              d['W3'].T, d['b3'].reshape(1, -1)]
    wlist = [w.astype(f32) for w in wlist]

    ia = jnp.transpose(atom_neighbor_list, (0, 2, 1)).reshape(B, K * A, 1)
    ib = jnp.transpose(bond_neighbor_list, (0, 2, 1)).reshape(B, K * A, 1)
    mask3 = atom_mask[..., None]

    in_specs = [
        pl.BlockSpec((MBLK, A, AF), lambda i: (i, 0, 0)),
        pl.BlockSpec((MBLK, MB, BF), lambda i: (i, 0, 0)),
        pl.BlockSpec((MBLK, K * A, 1), lambda i: (i, 0, 0)),
        pl.BlockSpec((MBLK, K * A, 1), lambda i: (i, 0, 0)),
        pl.BlockSpec((MBLK, A, K), lambda i: (i, 0, 0)),
        pl.BlockSpec((MBLK, A, 1), lambda i: (i, 0, 0)),
    ] + [pl.BlockSpec(wa.shape, lambda i, nd=wa.ndim: (0,) * nd)
         for wa in wlist]

    return pl.pallas_call(
        _body,
        grid=(B // MBLK,),
        in_specs=in_specs,
        out_specs=pl.BlockSpec((MBLK, 1), lambda i: (i, 0)),
        out_shape=jax.ShapeDtypeStruct((B, 1), f32),
    )(atom_features, bond_features, ia, ib, atom_neighbor_list, mask3,
      *wlist)


# speedup vs baseline: 17.4710x; 1.0979x over previous
import jax
import jax.numpy as jnp
from jax.experimental import pallas as pl

B, A, K, MB = 256, 64, 6, 64
AF, BF, FP = 72, 10, 128
NLAYERS, NMOL = 3, 2
EPS = 1e-06
NEG = -900000000.0
MBLK = 16
N = MBLK * A
_SQ = float((1.0 + EPS) ** 0.5)


def _lbn_w(p):
    return [p['W'].T, p['b'].reshape(1, -1),
            (p['g'] / _SQ).reshape(1, -1), p['be'].reshape(1, -1)]


def _leaky(x):
    return jnp.where(x >= 0, x, 0.01 * x)


def _elu(x):
    return jnp.where(x > 0, x, jnp.exp(jnp.minimum(x, 0.0)) - 1.0)


def _body(af_ref, bf_ref, ia_ref, ib_ref, ik_ref, mk_ref, *refs):
    o_ref = refs[-1]
    w = [r[...] for r in refs[:-1]]
    it = iter(w)

    def nxt(n_):
        return [next(it) for _ in range(n_)]

    f32 = jnp.float32
    bf16 = jnp.bfloat16
    dot = lambda a, b_: jnp.dot(a, b_, preferred_element_type=f32)

    def lbn(x, ws):
        Wt, b_, s_, be_ = ws
        return (dot(x, Wt) + b_) * s_ + be_

    x = af_ref[...].reshape(N, AF)
    x = jnp.maximum(lbn(x, nxt(4)), 0.0)
    atom_fp = jnp.maximum(lbn(x, nxt(4)), 0.0)
    y = bf_ref[...].reshape(N, BF)
    y = jnp.maximum(lbn(y, nxt(4)), 0.0)
    bond_fp = jnp.maximum(lbn(y, nxt(4)), 0.0)

    iota2 = jax.lax.broadcasted_iota(jnp.int32, (K * A, 2 * A), 1) & (A - 1)

    def build_oh(idx_all):
        return [(idx_all[m] == iota2).astype(bf16) for m in range(MBLK)]

    def gather_km(ohs, table):
        hi = table.astype(bf16)
        lo = (table - hi.astype(f32)).astype(bf16)
        outs = [[] for _ in range(K)]
        for m in range(MBLK):
            t2 = jnp.concatenate(
                [hi[m * A:(m + 1) * A], lo[m * A:(m + 1) * A]], axis=0)
            g = jnp.dot(ohs[m], t2, preferred_element_type=f32)
            for k in range(K):
                outs[k].append(g[k * A:(k + 1) * A])
        return [jnp.concatenate(c, axis=0) for c in outs]

    oh_a = build_oh(ia_ref[...])
    oh_b = build_oh(ib_ref[...])
    anei = gather_km(oh_a, atom_fp)
    bnei = gather_km(oh_b, bond_fp)

    nei_w1, nei_w2 = nxt(4), nxt(4)
    nei = []
    for k in range(K):
        a_k, b_k = anei[k], bnei[k]
        mix = a_k + b_k - a_k * b_k
        nb = jnp.concatenate([a_k, b_k, mix], axis=1)
        h1 = jnp.maximum(lbn(nb, nei_w1), 0.0)
        nei.append(jnp.maximum(lbn(h1, nei_w2), 0.0))

    row_k = jax.lax.broadcasted_iota(jnp.int32, (2 * K, K * FP), 0) % K
    col_k = jax.lax.broadcasted_iota(jnp.int32, (2 * K, K * FP), 1) // FP
    rep_mat = (row_k == col_k).astype(bf16)

    idxs = ik_ref[...].reshape(N, K)
    att_mask = (idxs != A - 1).astype(f32)
    smask = jnp.where(idxs == A - 1, NEG, 0.0).astype(f32)

    for _ in range(NLAYERS):
        (wal, Wbd, bal, sal, beal) = nxt(5)
        att_w = nxt(4)
        (Wih, bih, Whh, bhh) = nxt(4)
        nei_stack = jnp.concatenate(nei, axis=1)
        s = dot(atom_fp, wal) + dot(nei_stack, Wbd)
        s = (s + bal) * sal + beal
        s = _leaky(s) + smask
        s = s - jnp.max(s, axis=1, keepdims=True)
        e = jnp.exp(s)
        wgt = e / jnp.sum(e, axis=1, keepdims=True) * att_mask
        wsum = jnp.sum(wgt, axis=1, keepdims=True)
        whi = wgt.astype(bf16)
        wlo = (wgt - whi.astype(f32)).astype(bf16)
        wrep = jnp.dot(jnp.concatenate([whi, wlo], axis=1), rep_mat,
                       preferred_element_type=f32)
        weighted = wrep[:, :FP] * nei[0]
        for k in range(1, K):
            weighted = weighted + wrep[:, k * FP:(k + 1) * FP] * nei[k]
        Wt_a, b_a, s_a, be_a = att_w
        ctx = (dot(weighted, Wt_a) + wsum * b_a) * s_a + wsum * be_a
        ctx = _elu(ctx)
        gi = dot(ctx, Wih) + bih
        gh = dot(atom_fp, Whh) + bhh
        r = jax.nn.sigmoid(gi[:, :FP] + gh[:, :FP])
        z = jax.nn.sigmoid(gi[:, FP:2 * FP] + gh[:, FP:2 * FP])
        n = jnp.tanh(gi[:, 2 * FP:] + r * gh[:, 2 * FP:])
        new_atom = (1.0 - z) * n + z * atom_fp
        act = jnp.maximum(new_atom, 0.0)
        nei = gather_km(oh_a, act)
        atom_fp = new_atom

    mask3 = mk_ref[...]
    molmask = jnp.where(mask3 == 0.0, NEG, 0.0)
    afp3 = atom_fp.reshape(MBLK, A, FP)
    sup = jnp.sum(afp3 * mask3, axis=1)
    (wsm, wam, bam, sam, beam) = nxt(5)
    matt_w = nxt(4)
    (Wihm, bihm, Whhm, bhhm) = nxt(4)
    for _ in range(NMOL):
        s_at = dot(atom_fp, wam).reshape(MBLK, A, 1)
        s_sup = dot(sup, wsm).reshape(MBLK, 1, 1)
        s = (s_at + s_sup + bam) * sam + beam
        s = _leaky(s) + molmask
        s = s - jnp.max(s, axis=1, keepdims=True)
        e = jnp.exp(s)
        wgt = e / jnp.sum(e, axis=1, keepdims=True) * mask3
        att = lbn(atom_fp, matt_w)
        ctx = _elu(jnp.sum(att.reshape(MBLK, A, FP) * wgt, axis=1))
        gi = dot(ctx, Wihm) + bihm
        gh = dot(sup, Whhm) + bhhm
        r = jax.nn.sigmoid(gi[:, :FP] + gh[:, :FP])
        z = jax.nn.sigmoid(gi[:, FP:2 * FP] + gh[:, FP:2 * FP])
        n = jnp.tanh(gi[:, 2 * FP:] + r * gh[:, 2 * FP:])
        sup = (1.0 - z) * n + z * sup

    (W1d, b1d, W2d, b2d, W3d, b3d) = nxt(6)
    mol_fp = jnp.maximum(sup, 0.0)
    h = jnp.maximum(dot(mol_fp, W1d) + b1d, 0.0)
    h = jnp.maximum(dot(h, W2d) + b2d, 0.0)
    o_ref[...] = dot(h, W3d) + b3d


def kernel(atom_features, bond_features, atom_neighbor_list,
           bond_neighbor_list, atom_mask, params):
    f32 = jnp.float32
    wlist = []
    for p in params['atom_fc'] + params['bond_fc']:
        wlist += _lbn_w(p)
    for p in params['nei_fc']:
        wlist += _lbn_w(p)
    eye_k = jnp.eye(K, dtype=f32)
    for lp in params['layers']:
        al = lp['align']
        wt = al['W'].T
        wlist += [wt[:FP], jnp.kron(eye_k, wt[FP:]),
                  al['b'].reshape(1, 1), (al['g'] / _SQ).reshape(1, 1),
                  al['be'].reshape(1, 1)]
        wlist += _lbn_w(lp['attend'])
        g = lp['gru']
        wlist += [g['Wih'].T, g['bih'].reshape(1, -1),
                  g['Whh'].T, g['bhh'].reshape(1, -1)]
    mp = params['mol']
    al = mp['align']
    wt = al['W'].T
    wlist += [wt[:FP], wt[FP:], al['b'].reshape(1, 1),
              (al['g'] / _SQ).reshape(1, 1), al['be'].reshape(1, 1)]
    wlist += _lbn_w(mp['attend'])
    g = mp['gru']
    wlist += [g['Wih'].T, g['bih'].reshape(1, -1),
              g['Whh'].T, g['bhh'].reshape(1, -1)]
    d = params['dnn']
    wlist += [d['W1'].T, d['b1'].reshape(1, -1),
              d['W2'].T, d['b2'].reshape(1, -1),
              d['W3'].T, d['b3'].reshape(1, -1)]
    wlist = [w.astype(f32) for w in wlist]

    ia = jnp.transpose(atom_neighbor_list, (0, 2, 1)).reshape(B, K * A, 1)
    ib = jnp.transpose(bond_neighbor_list, (0, 2, 1)).reshape(B, K * A, 1)
    mask3 = atom_mask[..., None]

    in_specs = [
        pl.BlockSpec((MBLK, A, AF), lambda i: (i, 0, 0)),
        pl.BlockSpec((MBLK, MB, BF), lambda i: (i, 0, 0)),
        pl.BlockSpec((MBLK, K * A, 1), lambda i: (i, 0, 0)),
        pl.BlockSpec((MBLK, K * A, 1), lambda i: (i, 0, 0)),
        pl.BlockSpec((MBLK, A, K), lambda i: (i, 0, 0)),
        pl.BlockSpec((MBLK, A, 1), lambda i: (i, 0, 0)),
    ] + [pl.BlockSpec(wa.shape, lambda i, nd=wa.ndim: (0,) * nd)
         for wa in wlist]

    return pl.pallas_call(
        _body,
        grid=(B // MBLK,),
        in_specs=in_specs,
        out_specs=pl.BlockSpec((MBLK, 1), lambda i: (i, 0)),
        out_shape=jax.ShapeDtypeStruct((B, 1), f32),
    )(atom_features, bond_features, ia, ib, atom_neighbor_list, mask3,
      *wlist)


# speedup vs baseline: 18.5882x; 1.0639x over previous
import jax
import jax.numpy as jnp
from jax.experimental import pallas as pl

B, A, K, MB = 256, 64, 6, 64
AF, BF, FP = 72, 10, 128
NLAYERS, NMOL = 3, 2
EPS = 1e-06
NEG = -900000000.0
MBLK = 16
N = MBLK * A
_SQ = float((1.0 + EPS) ** 0.5)


def _lbn_w(p):
    return [p['W'].T, p['b'].reshape(1, -1),
            (p['g'] / _SQ).reshape(1, -1), p['be'].reshape(1, -1)]


def _leaky(x):
    return jnp.where(x >= 0, x, 0.01 * x)


def _elu(x):
    return jnp.where(x > 0, x, jnp.exp(jnp.minimum(x, 0.0)) - 1.0)


def _body(af_ref, bf_ref, ia_ref, ib_ref, ik_ref, mk_ref, *refs):
    o_ref = refs[-1]
    w = [r[...] for r in refs[:-1]]
    it = iter(w)

    def nxt(n_):
        return [next(it) for _ in range(n_)]

    f32 = jnp.float32
    bf16 = jnp.bfloat16
    dot = lambda a, b_: jnp.dot(a, b_, preferred_element_type=f32)

    def lbn(x, ws):
        Wt, b_, s_, be_ = ws
        return (dot(x, Wt) + b_) * s_ + be_

    x = af_ref[...].reshape(N, AF)
    x = jnp.maximum(lbn(x, nxt(4)), 0.0)
    atom_fp = jnp.maximum(lbn(x, nxt(4)), 0.0)
    y = bf_ref[...].reshape(N, BF)
    y = jnp.maximum(lbn(y, nxt(4)), 0.0)
    bond_fp = jnp.maximum(lbn(y, nxt(4)), 0.0)

    def gather_km(ohs, table):
        hi = table.astype(bf16)
        lo = (table - hi.astype(f32)).astype(bf16)
        outs = [[] for _ in range(K)]
        for m in range(MBLK):
            t2 = jnp.concatenate(
                [hi[m * A:(m + 1) * A], lo[m * A:(m + 1) * A]], axis=0)
            g = jnp.dot(ohs[m], t2, preferred_element_type=f32)
            for k in range(K):
                outs[k].append(g[k * A:(k + 1) * A])
        return [jnp.concatenate(c, axis=0) for c in outs]

    oh_a = [ia_ref[m] for m in range(MBLK)]
    oh_b = [ib_ref[m] for m in range(MBLK)]
    anei = gather_km(oh_a, atom_fp)
    bnei = gather_km(oh_b, bond_fp)

    nei_w1, nei_w2 = nxt(4), nxt(4)
    nei = []
    for k in range(K):
        a_k, b_k = anei[k], bnei[k]
        mix = a_k + b_k - a_k * b_k
        nb = jnp.concatenate([a_k, b_k, mix], axis=1)
        h1 = jnp.maximum(lbn(nb, nei_w1), 0.0)
        nei.append(jnp.maximum(lbn(h1, nei_w2), 0.0))

    row_k = jax.lax.broadcasted_iota(jnp.int32, (2 * K, K * FP), 0) % K
    col_k = jax.lax.broadcasted_iota(jnp.int32, (2 * K, K * FP), 1) // FP
    rep_mat = (row_k == col_k).astype(bf16)

    idxs = ik_ref[...].reshape(N, K)
    att_mask = (idxs != A - 1).astype(f32)
    smask = jnp.where(idxs == A - 1, NEG, 0.0).astype(f32)

    for _ in range(NLAYERS):
        (wal, Wbd, bal, sal, beal) = nxt(5)
        att_w = nxt(4)
        (Wih, bih, Whh, bhh) = nxt(4)
        nei_stack = jnp.concatenate(nei, axis=1)
        s = dot(atom_fp, wal) + dot(nei_stack, Wbd)
        s = (s + bal) * sal + beal
        s = _leaky(s) + smask
        s = s - jnp.max(s, axis=1, keepdims=True)
        e = jnp.exp(s)
        wgt = e / jnp.sum(e, axis=1, keepdims=True) * att_mask
        whi = wgt.astype(bf16)
        wlo = (wgt - whi.astype(f32)).astype(bf16)
        wrep = jnp.dot(jnp.concatenate([whi, wlo], axis=1), rep_mat,
                       preferred_element_type=f32)
        Wt_a, b_a, s_a, be_a = att_w
        acc = None
        for k in range(K):
            att_k = (dot(nei[k], Wt_a) + b_a) * s_a + be_a
            term = wrep[:, k * FP:(k + 1) * FP] * att_k
            acc = term if acc is None else acc + term
        ctx = _elu(acc)
        gi = dot(ctx, Wih) + bih
        gh = dot(atom_fp, Whh) + bhh
        r = jax.nn.sigmoid(gi[:, :FP] + gh[:, :FP])
        z = jax.nn.sigmoid(gi[:, FP:2 * FP] + gh[:, FP:2 * FP])
        n = jnp.tanh(gi[:, 2 * FP:] + r * gh[:, 2 * FP:])
        new_atom = (1.0 - z) * n + z * atom_fp
        act = jnp.maximum(new_atom, 0.0)
        nei = gather_km(oh_a, act)
        atom_fp = new_atom

    mask3 = mk_ref[...]
    molmask = jnp.where(mask3 == 0.0, NEG, 0.0)
    afp3 = atom_fp.reshape(MBLK, A, FP)
    sup = jnp.sum(afp3 * mask3, axis=1)
    (wsm, wam, bam, sam, beam) = nxt(5)
    matt_w = nxt(4)
    (Wihm, bihm, Whhm, bhhm) = nxt(4)
    for _ in range(NMOL):
        s_at = dot(atom_fp, wam).reshape(MBLK, A, 1)
        s_sup = dot(sup, wsm).reshape(MBLK, 1, 1)
        s = (s_at + s_sup + bam) * sam + beam
        s = _leaky(s) + molmask
        s = s - jnp.max(s, axis=1, keepdims=True)
        e = jnp.exp(s)
        wgt = e / jnp.sum(e, axis=1, keepdims=True) * mask3
        att = lbn(atom_fp, matt_w)
        ctx = _elu(jnp.sum(att.reshape(MBLK, A, FP) * wgt, axis=1))
        gi = dot(ctx, Wihm) + bihm
        gh = dot(sup, Whhm) + bhhm
        r = jax.nn.sigmoid(gi[:, :FP] + gh[:, :FP])
        z = jax.nn.sigmoid(gi[:, FP:2 * FP] + gh[:, FP:2 * FP])
        n = jnp.tanh(gi[:, 2 * FP:] + r * gh[:, 2 * FP:])
        sup = (1.0 - z) * n + z * sup

    (W1d, b1d, W2d, b2d, W3d, b3d) = nxt(6)
    mol_fp = jnp.maximum(sup, 0.0)
    h = jnp.maximum(dot(mol_fp, W1d) + b1d, 0.0)
    h = jnp.maximum(dot(h, W2d) + b2d, 0.0)
    o_ref[...] = dot(h, W3d) + b3d


def kernel(atom_features, bond_features, atom_neighbor_list,
           bond_neighbor_list, atom_mask, params):
    f32 = jnp.float32
    wlist = []
    for p in params['atom_fc'] + params['bond_fc']:
        wlist += _lbn_w(p)
    for p in params['nei_fc']:
        wlist += _lbn_w(p)
    eye_k = jnp.eye(K, dtype=f32)
    for lp in params['layers']:
        al = lp['align']
        wt = al['W'].T
        wlist += [wt[:FP], jnp.kron(eye_k, wt[FP:]),
                  al['b'].reshape(1, 1), (al['g'] / _SQ).reshape(1, 1),
                  al['be'].reshape(1, 1)]
        wlist += _lbn_w(lp['attend'])
        g = lp['gru']
        wlist += [g['Wih'].T, g['bih'].reshape(1, -1),
                  g['Whh'].T, g['bhh'].reshape(1, -1)]
    mp = params['mol']
    al = mp['align']
    wt = al['W'].T
    wlist += [wt[:FP], wt[FP:], al['b'].reshape(1, 1),
              (al['g'] / _SQ).reshape(1, 1), al['be'].reshape(1, 1)]
    wlist += _lbn_w(mp['attend'])
    g = mp['gru']
    wlist += [g['Wih'].T, g['bih'].reshape(1, -1),
              g['Whh'].T, g['bhh'].reshape(1, -1)]
    d = params['dnn']
    wlist += [d['W1'].T, d['b1'].reshape(1, -1),
              d['W2'].T, d['b2'].reshape(1, -1),
              d['W3'].T, d['b3'].reshape(1, -1)]
    wlist = [w.astype(f32) for w in wlist]

    lane_val = (jnp.arange(2 * A, dtype=jnp.int32) & (A - 1))
    ia = jnp.transpose(atom_neighbor_list, (0, 2, 1)).reshape(B, K * A)
    ib = jnp.transpose(bond_neighbor_list, (0, 2, 1)).reshape(B, K * A)
    oh_a = (ia[:, :, None] == lane_val).astype(jnp.bfloat16)
    oh_b = (ib[:, :, None] == lane_val).astype(jnp.bfloat16)
    mask3 = atom_mask[..., None]

    in_specs = [
        pl.BlockSpec((MBLK, A, AF), lambda i: (i, 0, 0)),
        pl.BlockSpec((MBLK, MB, BF), lambda i: (i, 0, 0)),
        pl.BlockSpec((MBLK, K * A, 2 * A), lambda i: (i, 0, 0)),
        pl.BlockSpec((MBLK, K * A, 2 * A), lambda i: (i, 0, 0)),
        pl.BlockSpec((MBLK, A, K), lambda i: (i, 0, 0)),
        pl.BlockSpec((MBLK, A, 1), lambda i: (i, 0, 0)),
    ] + [pl.BlockSpec(wa.shape, lambda i, nd=wa.ndim: (0,) * nd)
         for wa in wlist]

    return pl.pallas_call(
        _body,
        grid=(B // MBLK,),
        in_specs=in_specs,
        out_specs=pl.BlockSpec((MBLK, 1), lambda i: (i, 0)),
        out_shape=jax.ShapeDtypeStruct((B, 1), f32),
    )(atom_features, bond_features, oh_a, oh_b, atom_neighbor_list, mask3,
      *wlist)
